# baseline (device time: 46567 ns/iter reference)
import jax
import jax.numpy as jnp
from jax import lax
from jax.experimental import pallas as pl
from jax.experimental.pallas import tpu as pltpu

N_DEV = 16


def kernel(x, w_mat):
    m_per, k = x.shape
    _, n = w_mat.shape
    n_per = n // N_DEV

    def body(x_ref, w_hbm, out_ref, w_bufs, send_bufs, w_sems, send_sems, recv_sems):
        my = lax.axis_index("i")

        def w_chunk_copy(t):
            j = (my + 1 + t) % N_DEV
            return pltpu.make_async_copy(
                w_hbm.at[:, pl.ds(j * n_per, n_per)],
                w_bufs.at[t % 2],
                w_sems.at[t % 2],
            )

        x_val = x_ref[...]
        w_chunk_copy(0).start()

        rdmas = []
        for t in range(N_DEV):
            if t + 1 < N_DEV:
                w_chunk_copy(t + 1).start()
            w_chunk_copy(t).wait()
            blk = jnp.maximum(
                jnp.dot(x_val, w_bufs[t % 2], preferred_element_type=jnp.float32),
                0.0,
            )
            if t < N_DEV - 1:
                s = t + 1
                j = (my + 1 + t) % N_DEV
                send_bufs[s] = blk
                rdma = pltpu.make_async_remote_copy(
                    src_ref=send_bufs.at[s],
                    dst_ref=out_ref.at[pl.ds(my * m_per, m_per), :],
                    send_sem=send_sems.at[s],
                    recv_sem=recv_sems.at[s],
                    device_id=(j,),
                    device_id_type=pl.DeviceIdType.MESH,
                )
                rdma.start()
                rdmas.append(rdma)
            else:
                out_ref[pl.ds(my * m_per, m_per), :] = blk

        for s in range(1, N_DEV):
            rdmas[s - 1].wait_recv()
        for s in range(1, N_DEV):
            rdmas[s - 1].wait_send()

    return pl.pallas_call(
        body,
        out_shape=jax.ShapeDtypeStruct((N_DEV * m_per, n_per), jnp.float32),
        in_specs=[
            pl.BlockSpec(memory_space=pltpu.VMEM),
            pl.BlockSpec(memory_space=pl.ANY),
        ],
        out_specs=pl.BlockSpec(memory_space=pltpu.VMEM),
        scratch_shapes=[
            pltpu.VMEM((2, k, n_per), jnp.float32),
            pltpu.VMEM((N_DEV, m_per, n_per), jnp.float32),
            pltpu.SemaphoreType.DMA((2,)),
            pltpu.SemaphoreType.DMA((N_DEV,)),
            pltpu.SemaphoreType.DMA((N_DEV,)),
        ],
        compiler_params=pltpu.CompilerParams(
            vmem_limit_bytes=100 * 1024 * 1024,
        ),
    )(x, w_mat)
